# baseline (speedup 1.0000x reference)
import functools

import jax
import jax.numpy as jnp
from jax import lax
from jax.experimental import pallas as pl
from jax.experimental.pallas import tpu as pltpu
from jax.experimental.pallas import tpu_sc as plsc

B, N, T, D, K = 8, 2048, 5, 64, 21
RB = 256
NI = N // RB
CNT = float(B * N)
GW = 768
KP = 24
NC, NS, L = 2, 16, 16
NW = NC * NS
RPW = N // NW
RC = 2


def _a_body(xtb_ref, emb_ref, embb_ref, nrow_ref, ncol_ref, W_ref, bv_ref,
            al_ref, ar_ref, idx_ref, ger_ref):
    emb = emb_ref[...]
    embb = embb_ref[...]
    raw = lax.dot_general(embb, emb, (((1,), (1,)), ((), ())),
                          preferred_element_type=jnp.float32)
    cos = raw / (ncol_ref[...] * nrow_ref[...])

    iota = lax.broadcasted_iota(jnp.int32, (RB, N), 1).astype(jnp.float32)
    work = cos
    idxs = []
    for t in range(K):
        m = jnp.max(work, axis=1, keepdims=True)
        sel = work == m
        amx = jnp.min(jnp.where(sel, iota, jnp.float32(3e8)),
                      axis=1, keepdims=True)
        idxs.append(amx)
        if t < K - 1:
            work = jnp.where(iota == amx, jnp.float32(-1e30), work)
    idxs += [idxs[0]] * (KP - K)
    idx_ref[...] = jnp.concatenate(idxs, axis=1).astype(jnp.int32)

    W = W_ref[...]
    bv = bv_ref[...]
    al1, al2 = al_ref[0:1, :], al_ref[1:2, :]
    ar1, ar2 = ar_ref[0:1, :], ar_ref[1:2, :]
    el_embb = jnp.sum(embb * al2, axis=1, keepdims=True)
    er_embb = jnp.sum(embb * ar2, axis=1, keepdims=True)
    els = []
    ers = []
    for bb in range(B):
        xbb = xtb_ref[bb]
        gb = lax.dot_general(xbb, W, (((0,), (1,)), ((), ())),
                             preferred_element_type=jnp.float32) + bv
        ger_ref[:, bb * D:(bb + 1) * D] = gb
        els.append(jnp.sum(gb * al1, axis=1, keepdims=True) + el_embb)
        ers.append(jnp.sum(gb * ar1, axis=1, keepdims=True) + er_embb)
    el8 = jnp.concatenate(els, axis=1)
    er8 = jnp.concatenate(ers, axis=1)
    q = (lax.broadcasted_iota(jnp.int32, (B, 2 * D), 1) // L
         == lax.broadcasted_iota(jnp.int32, (B, 2 * D), 0)).astype(jnp.float32)
    ger_ref[:, B * D:B * D + 2 * D] = lax.dot_general(
        er8, q, (((1,), (0,)), ((), ())),
        preferred_element_type=jnp.float32,
        precision=jax.lax.Precision.HIGHEST)
    ger_ref[:, B * D + 2 * D:GW] = lax.dot_general(
        el8, q, (((1,), (0,)), ((), ())),
        preferred_element_type=jnp.float32,
        precision=jax.lax.Precision.HIGHEST)


def _sc_body(idx_hbm, g_hbm, out_hbm, idx_v, own_v, gb0, gb1, out_v,
             sem0, sem1):
    wid = lax.axis_index("s") * NC + lax.axis_index("c")
    base = wid * RPW
    pltpu.sync_copy(idx_hbm.at[pl.ds(base, RPW)], idx_v)
    pltpu.async_copy(g_hbm.at[idx_v.at[0]], gb0, sem0)

    def chunk_body(c, carry):
        r0 = c * RC
        pltpu.sync_copy(g_hbm.at[pl.ds(base + r0, RC)], own_v)
        for rr in range(RC):
            r = r0 + rr
            gbuf, sem = (gb0, sem0) if rr == 0 else (gb1, sem1)
            nbuf, nsem = (gb1, sem1) if rr == 0 else (gb0, sem0)
            rn = jnp.minimum(r + 1, RPW - 1)
            pltpu.async_copy(g_hbm.at[idx_v.at[rn]], nbuf, nsem)
            pltpu.make_async_copy(g_hbm.at[idx_v.at[r]], gbuf, sem).wait()
            for bb in range(B):
                elsv = own_v[rr, pl.ds(B * D + 2 * D + bb * L, L)]
                ls = []
                m = jnp.full((L,), -1e9, jnp.float32)
                for jj in range(K):
                    l = elsv + gbuf[jj, pl.ds(B * D + bb * L, L)]
                    l = jnp.maximum(l, 0.2 * l)
                    ls.append(l)
                    m = jnp.maximum(m, l)
                den = jnp.zeros((L,), jnp.float32)
                acc = [jnp.zeros((L,), jnp.float32) for _ in range(4)]
                for jj in range(K):
                    w = jnp.exp(ls[jj] - m)
                    den = den + w
                    for dc in range(4):
                        acc[dc] = acc[dc] + w * gbuf[jj, pl.ds(bb * D + dc * L, L)]
                winv = 1.0 / den
                for dc in range(4):
                    out_v[rr, pl.ds(bb * D + dc * L, L)] = acc[dc] * winv
        pltpu.sync_copy(out_v, out_hbm.at[pl.ds(base + r0, RC)])
        return carry

    lax.fori_loop(0, RPW // RC, chunk_body, 0)
    pltpu.make_async_copy(g_hbm.at[idx_v.at[RPW - 1]], gb0, sem0).wait()


def _c_body(op_ref, embb_ref, gam_ref, bet_ref, fw_ref, fb_ref, y_ref, s_ref):
    i = pl.program_id(0)
    ph = i // NI

    @pl.when(i == 0)
    def _():
        s_ref[...] = jnp.zeros_like(s_ref)

    z = op_ref[...]
    embb = embb_ref[...]
    e = (lax.broadcasted_iota(jnp.int32, (D, B * D), 1) % D
         == lax.broadcasted_iota(jnp.int32, (D, B * D), 0)).astype(jnp.float32)
    embdup = lax.dot_general(embb, e, (((1,), (0,)), ((), ())),
                             preferred_element_type=jnp.float32,
                             precision=jax.lax.Precision.HIGHEST)
    op = jnp.maximum(z, 0.0) * embdup

    @pl.when(ph == 0)
    def _():
        s_ref[0:1, :] += jnp.sum(op, axis=0, keepdims=True)
        s_ref[1:2, :] += jnp.sum(op * op, axis=0, keepdims=True)

    @pl.when(ph == 1)
    def _():
        t1 = jnp.zeros((1, D), jnp.float32)
        t2 = jnp.zeros((1, D), jnp.float32)
        for bb in range(B):
            t1 = t1 + s_ref[0:1, bb * D:(bb + 1) * D]
            t2 = t2 + s_ref[1:2, bb * D:(bb + 1) * D]
        mean = t1 * (1.0 / CNT)
        var = t2 * (1.0 / CNT) - mean * mean
        scale = lax.rsqrt(var + 1e-5) * gam_ref[...]
        shift = bet_ref[...] - mean * scale
        fw = fw_ref[...]
        fb = fb_ref[...]
        for bb in range(B):
            o = op[:, bb * D:(bb + 1) * D] * scale + shift
            o = jnp.maximum(o, 0.0)
            r = lax.dot_general(fw, o, (((1,), (1,)), ((), ())),
                                preferred_element_type=jnp.float32)
            y_ref[bb:bb + 1, :] = r + fb


_sc_mesh = plsc.VectorSubcoreMesh(core_axis_name="c", subcore_axis_name="s")

_sc_attn = functools.partial(
    pl.kernel,
    mesh=_sc_mesh,
    out_type=jax.ShapeDtypeStruct((N, B * D), jnp.float32),
    scratch_types=[
        pltpu.VMEM((RPW, KP), jnp.int32),
        pltpu.VMEM((RC, GW), jnp.float32),
        pltpu.VMEM((KP, GW), jnp.float32),
        pltpu.VMEM((KP, GW), jnp.float32),
        pltpu.VMEM((RC, B * D), jnp.float32),
        pltpu.SemaphoreType.DMA,
        pltpu.SemaphoreType.DMA,
    ],
)(_sc_body)


def kernel(x, emb_table, W, b, a_l, a_r, bn_gamma, bn_beta, fc_w, fc_b):
    xt = jnp.transpose(x, (0, 2, 1))
    bv = b.reshape(1, D)
    al = a_l.reshape(2, D)
    ar = a_r.reshape(2, D)
    nn = jnp.linalg.norm(emb_table, axis=-1)
    nrow = nn.reshape(1, N)
    ncol = nn.reshape(N, 1)

    idx, ger = pl.pallas_call(
        _a_body,
        grid=(NI,),
        in_specs=[
            pl.BlockSpec((B, T, RB), lambda i: (0, 0, i)),
            pl.BlockSpec((N, D), lambda i: (0, 0)),
            pl.BlockSpec((RB, D), lambda i: (i, 0)),
            pl.BlockSpec((1, N), lambda i: (0, 0)),
            pl.BlockSpec((RB, 1), lambda i: (i, 0)),
            pl.BlockSpec((D, T), lambda i: (0, 0)),
            pl.BlockSpec((1, D), lambda i: (0, 0)),
            pl.BlockSpec((2, D), lambda i: (0, 0)),
            pl.BlockSpec((2, D), lambda i: (0, 0)),
        ],
        out_specs=[
            pl.BlockSpec((RB, KP), lambda i: (i, 0)),
            pl.BlockSpec((RB, GW), lambda i: (i, 0)),
        ],
        out_shape=[
            jax.ShapeDtypeStruct((N, KP), jnp.int32),
            jax.ShapeDtypeStruct((N, GW), jnp.float32),
        ],
        compiler_params=pltpu.CompilerParams(
            dimension_semantics=("arbitrary",)),
    )(xt, emb_table, emb_table, nrow, ncol, W, bv, al, ar)

    z = _sc_attn(idx, ger)

    y = pl.pallas_call(
        _c_body,
        grid=(2 * NI,),
        in_specs=[
            pl.BlockSpec((RB, B * D), lambda i: (i % NI, 0)),
            pl.BlockSpec((RB, D), lambda i: (i % NI, 0)),
            pl.BlockSpec((1, D), lambda i: (0, 0)),
            pl.BlockSpec((1, D), lambda i: (0, 0)),
            pl.BlockSpec((1, D), lambda i: (0, 0)),
            pl.BlockSpec((1, 1), lambda i: (0, 0)),
        ],
        out_specs=pl.BlockSpec((B, RB), lambda i: (0, i % NI)),
        out_shape=jax.ShapeDtypeStruct((B, N), jnp.float32),
        scratch_shapes=[pltpu.VMEM((2, B * D), jnp.float32)],
        compiler_params=pltpu.CompilerParams(
            dimension_semantics=("arbitrary",)),
    )(z, emb_table, bn_gamma.reshape(1, D), bn_beta.reshape(1, D),
      fc_w.reshape(1, D), fc_b.reshape(1, 1))
    return y


# speedup vs baseline: 1.0210x; 1.0210x over previous
import functools

import jax
import jax.numpy as jnp
from jax import lax
from jax.experimental import pallas as pl
from jax.experimental.pallas import tpu as pltpu
from jax.experimental.pallas import tpu_sc as plsc

B, N, T, D, K = 8, 2048, 5, 64, 21
RB = 256
NI = N // RB
CNT = float(B * N)
GW = 768
KP = 21
NC, NS, L = 2, 16, 16
NW = NC * NS
RPW = N // NW
RC = 2


def _a_body(xtb_ref, emb_ref, embb_ref, nrow_ref, ncol_ref, W_ref, bv_ref,
            al_ref, ar_ref, idx_ref, ger_ref):
    emb = emb_ref[...]
    embb = embb_ref[...]
    raw = lax.dot_general(embb, emb, (((1,), (1,)), ((), ())),
                          preferred_element_type=jnp.float32)
    cos = raw / (ncol_ref[...] * nrow_ref[...])

    iota = lax.broadcasted_iota(jnp.int32, (RB, N), 1).astype(jnp.float32)
    work = cos
    idxs = []
    for t in range(K):
        m = jnp.max(work, axis=1, keepdims=True)
        sel = work == m
        amx = jnp.min(jnp.where(sel, iota, jnp.float32(3e8)),
                      axis=1, keepdims=True)
        idxs.append(amx)
        if t < K - 1:
            work = jnp.where(iota == amx, jnp.float32(-1e30), work)
    idxs += [idxs[0]] * (KP - K)
    idx_ref[...] = jnp.concatenate(idxs, axis=1).astype(jnp.int32)

    W = W_ref[...]
    bv = bv_ref[...]
    al1, al2 = al_ref[0:1, :], al_ref[1:2, :]
    ar1, ar2 = ar_ref[0:1, :], ar_ref[1:2, :]
    el_embb = jnp.sum(embb * al2, axis=1, keepdims=True)
    er_embb = jnp.sum(embb * ar2, axis=1, keepdims=True)
    els = []
    ers = []
    for bb in range(B):
        xbb = xtb_ref[bb]
        gb = lax.dot_general(xbb, W, (((0,), (1,)), ((), ())),
                             preferred_element_type=jnp.float32) + bv
        ger_ref[:, bb * D:(bb + 1) * D] = gb
        els.append(jnp.sum(gb * al1, axis=1, keepdims=True) + el_embb)
        ers.append(jnp.sum(gb * ar1, axis=1, keepdims=True) + er_embb)
    el8 = jnp.concatenate(els, axis=1)
    er8 = jnp.concatenate(ers, axis=1)
    q = (lax.broadcasted_iota(jnp.int32, (B, 2 * D), 1) // L
         == lax.broadcasted_iota(jnp.int32, (B, 2 * D), 0)).astype(jnp.float32)
    ger_ref[:, B * D:B * D + 2 * D] = lax.dot_general(
        er8, q, (((1,), (0,)), ((), ())),
        preferred_element_type=jnp.float32,
        precision=jax.lax.Precision.HIGHEST)
    ger_ref[:, B * D + 2 * D:GW] = lax.dot_general(
        el8, q, (((1,), (0,)), ((), ())),
        preferred_element_type=jnp.float32,
        precision=jax.lax.Precision.HIGHEST)


def _sc_body(idx_hbm, g_hbm, out_hbm, idx_v, own_v, gb0, gb1, out_v,
             sem0, sem1):
    wid = lax.axis_index("s") * NC + lax.axis_index("c")
    base = wid * RPW
    pltpu.sync_copy(idx_hbm.at[pl.ds(base, RPW)], idx_v)
    pltpu.async_copy(g_hbm.at[idx_v.at[0]], gb0, sem0)

    def chunk_body(c, carry):
        r0 = c * RC
        pltpu.sync_copy(g_hbm.at[pl.ds(base + r0, RC)], own_v)
        for rr in range(RC):
            r = r0 + rr
            gbuf, sem = (gb0, sem0) if rr == 0 else (gb1, sem1)
            nbuf, nsem = (gb1, sem1) if rr == 0 else (gb0, sem0)
            rn = jnp.minimum(r + 1, RPW - 1)
            pltpu.async_copy(g_hbm.at[idx_v.at[rn]], nbuf, nsem)
            pltpu.make_async_copy(g_hbm.at[idx_v.at[r]], gbuf, sem).wait()
            for bb in range(B):
                elsv = own_v[rr, pl.ds(B * D + 2 * D + bb * L, L)]
                ls = []
                m = jnp.full((L,), -1e9, jnp.float32)
                for jj in range(K):
                    l = elsv + gbuf[jj, pl.ds(B * D + bb * L, L)]
                    l = jnp.maximum(l, 0.2 * l)
                    ls.append(l)
                    m = jnp.maximum(m, l)
                den = jnp.zeros((L,), jnp.float32)
                acc = [jnp.zeros((L,), jnp.float32) for _ in range(4)]
                for jj in range(K):
                    w = jnp.exp(ls[jj] - m)
                    den = den + w
                    for dc in range(4):
                        acc[dc] = acc[dc] + w * gbuf[jj, pl.ds(bb * D + dc * L, L)]
                winv = 1.0 / den
                for dc in range(4):
                    out_v[rr, pl.ds(bb * D + dc * L, L)] = acc[dc] * winv
        pltpu.sync_copy(out_v, out_hbm.at[pl.ds(base + r0, RC)])
        return carry

    lax.fori_loop(0, RPW // RC, chunk_body, 0)
    pltpu.make_async_copy(g_hbm.at[idx_v.at[RPW - 1]], gb0, sem0).wait()


def _c_body(op_ref, embb_ref, gam_ref, bet_ref, fw_ref, fb_ref, y_ref, s_ref):
    i = pl.program_id(0)
    ph = i // NI

    @pl.when(i == 0)
    def _():
        s_ref[...] = jnp.zeros_like(s_ref)

    z = op_ref[...]
    embb = embb_ref[...]
    e = (lax.broadcasted_iota(jnp.int32, (D, B * D), 1) % D
         == lax.broadcasted_iota(jnp.int32, (D, B * D), 0)).astype(jnp.float32)
    embdup = lax.dot_general(embb, e, (((1,), (0,)), ((), ())),
                             preferred_element_type=jnp.float32,
                             precision=jax.lax.Precision.HIGHEST)
    op = jnp.maximum(z, 0.0) * embdup

    @pl.when(ph == 0)
    def _():
        s_ref[0:1, :] += jnp.sum(op, axis=0, keepdims=True)
        s_ref[1:2, :] += jnp.sum(op * op, axis=0, keepdims=True)

    @pl.when(ph == 1)
    def _():
        t1 = jnp.zeros((1, D), jnp.float32)
        t2 = jnp.zeros((1, D), jnp.float32)
        for bb in range(B):
            t1 = t1 + s_ref[0:1, bb * D:(bb + 1) * D]
            t2 = t2 + s_ref[1:2, bb * D:(bb + 1) * D]
        mean = t1 * (1.0 / CNT)
        var = t2 * (1.0 / CNT) - mean * mean
        scale = lax.rsqrt(var + 1e-5) * gam_ref[...]
        shift = bet_ref[...] - mean * scale
        fw = fw_ref[...]
        fb = fb_ref[...]
        for bb in range(B):
            o = op[:, bb * D:(bb + 1) * D] * scale + shift
            o = jnp.maximum(o, 0.0)
            r = lax.dot_general(fw, o, (((1,), (1,)), ((), ())),
                                preferred_element_type=jnp.float32)
            y_ref[bb:bb + 1, :] = r + fb


_sc_mesh = plsc.VectorSubcoreMesh(core_axis_name="c", subcore_axis_name="s")

_sc_attn = functools.partial(
    pl.kernel,
    mesh=_sc_mesh,
    out_type=jax.ShapeDtypeStruct((N, B * D), jnp.float32),
    scratch_types=[
        pltpu.VMEM((RPW, KP), jnp.int32),
        pltpu.VMEM((RC, GW), jnp.float32),
        pltpu.VMEM((KP, GW), jnp.float32),
        pltpu.VMEM((KP, GW), jnp.float32),
        pltpu.VMEM((RC, B * D), jnp.float32),
        pltpu.SemaphoreType.DMA,
        pltpu.SemaphoreType.DMA,
    ],
)(_sc_body)


def kernel(x, emb_table, W, b, a_l, a_r, bn_gamma, bn_beta, fc_w, fc_b):
    xt = jnp.transpose(x, (0, 2, 1))
    bv = b.reshape(1, D)
    al = a_l.reshape(2, D)
    ar = a_r.reshape(2, D)
    nn = jnp.linalg.norm(emb_table, axis=-1)
    nrow = nn.reshape(1, N)
    ncol = nn.reshape(N, 1)

    idx, ger = pl.pallas_call(
        _a_body,
        grid=(NI,),
        in_specs=[
            pl.BlockSpec((B, T, RB), lambda i: (0, 0, i)),
            pl.BlockSpec((N, D), lambda i: (0, 0)),
            pl.BlockSpec((RB, D), lambda i: (i, 0)),
            pl.BlockSpec((1, N), lambda i: (0, 0)),
            pl.BlockSpec((RB, 1), lambda i: (i, 0)),
            pl.BlockSpec((D, T), lambda i: (0, 0)),
            pl.BlockSpec((1, D), lambda i: (0, 0)),
            pl.BlockSpec((2, D), lambda i: (0, 0)),
            pl.BlockSpec((2, D), lambda i: (0, 0)),
        ],
        out_specs=[
            pl.BlockSpec((RB, KP), lambda i: (i, 0)),
            pl.BlockSpec((RB, GW), lambda i: (i, 0)),
        ],
        out_shape=[
            jax.ShapeDtypeStruct((N, KP), jnp.int32),
            jax.ShapeDtypeStruct((N, GW), jnp.float32),
        ],
        compiler_params=pltpu.CompilerParams(
            dimension_semantics=("arbitrary",)),
    )(xt, emb_table, emb_table, nrow, ncol, W, bv, al, ar)

    z = _sc_attn(idx, ger)

    y = pl.pallas_call(
        _c_body,
        grid=(2 * NI,),
        in_specs=[
            pl.BlockSpec((RB, B * D), lambda i: (i % NI, 0)),
            pl.BlockSpec((RB, D), lambda i: (i % NI, 0)),
            pl.BlockSpec((1, D), lambda i: (0, 0)),
            pl.BlockSpec((1, D), lambda i: (0, 0)),
            pl.BlockSpec((1, D), lambda i: (0, 0)),
            pl.BlockSpec((1, 1), lambda i: (0, 0)),
        ],
        out_specs=pl.BlockSpec((B, RB), lambda i: (0, i % NI)),
        out_shape=jax.ShapeDtypeStruct((B, N), jnp.float32),
        scratch_shapes=[pltpu.VMEM((2, B * D), jnp.float32)],
        compiler_params=pltpu.CompilerParams(
            dimension_semantics=("arbitrary",)),
    )(z, emb_table, bn_gamma.reshape(1, D), bn_beta.reshape(1, D),
      fc_w.reshape(1, D), fc_b.reshape(1, 1))
    return y
